# baseline (speedup 1.0000x reference)
import functools

import numpy as np
import jax
import jax.numpy as jnp
from jax import lax
from jax.experimental import pallas as pl
from jax.experimental.pallas import tpu as pltpu
from jax.experimental.pallas import tpu_sc as plsc

_B, _N = 4096, 8192
_NC, _NS = 2, 16
_NW = _NC * _NS
_L = 16

_SLAB_ROWS = 8
_NSLABS = _B // _SLAB_ROWS
_SLABS_PER_W = _NSLABS // _NW
_CH = 2048
_NCHUNK = _N // _CH
_CHW = _CH // 2

_perms_cache = None

_U32 = np.uint32


def _rotl(x, d):
    return (x << _U32(d)) | (x >> _U32(32 - d))


def _threefry2x32(k1, k2, x1, x2):
    rot0 = (13, 15, 26, 6)
    rot1 = (17, 29, 16, 24)
    ks0, ks1 = _U32(k1), _U32(k2)
    ks2 = ks0 ^ ks1 ^ _U32(0x1BD11BDA)
    v = [(x1 + ks0).astype(_U32), (x2 + ks1).astype(_U32)]

    def rounds(rots):
        for r in rots:
            v[0] = (v[0] + v[1]).astype(_U32)
            v[1] = v[0] ^ _rotl(v[1], r)

    rounds(rot0); v[0] = (v[0] + ks1).astype(_U32); v[1] = (v[1] + ks2 + _U32(1)).astype(_U32)
    rounds(rot1); v[0] = (v[0] + ks2).astype(_U32); v[1] = (v[1] + ks0 + _U32(2)).astype(_U32)
    rounds(rot0); v[0] = (v[0] + ks0).astype(_U32); v[1] = (v[1] + ks1 + _U32(3)).astype(_U32)
    rounds(rot1); v[0] = (v[0] + ks1).astype(_U32); v[1] = (v[1] + ks2 + _U32(4)).astype(_U32)
    rounds(rot0); v[0] = (v[0] + ks2).astype(_U32); v[1] = (v[1] + ks0 + _U32(5)).astype(_U32)
    return v[0], v[1]


def _perms() -> np.ndarray:
    global _perms_cache
    if _perms_cache is not None:
        return _perms_cache
    seed = 42
    b1, b2 = _threefry2x32(_U32(seed >> 32), _U32(seed & 0xFFFFFFFF),
                           np.zeros(_B, dtype=_U32), np.arange(_B, dtype=_U32))
    keys = np.stack([b1, b2], axis=1)

    perm = np.broadcast_to(np.arange(_N, dtype=np.int32), (_B, _N)).copy()
    num_rounds = int(np.ceil(3 * np.log(max(1, _N)) / np.log(np.iinfo(np.uint32).max)))
    z2 = np.broadcast_to(np.zeros(2, dtype=_U32), (_B, 2))
    i2 = np.broadcast_to(np.arange(2, dtype=_U32), (_B, 2))
    zN = np.broadcast_to(np.zeros(_N, dtype=_U32), (_B, _N))
    iN = np.broadcast_to(np.arange(_N, dtype=_U32), (_B, _N))
    for _ in range(num_rounds):
        s1, s2 = _threefry2x32(keys[:, 0, None], keys[:, 1, None], z2, i2)
        keys = np.stack([s1[:, 0], s2[:, 0]], axis=1)
        r1, r2 = _threefry2x32(s1[:, 1, None], s2[:, 1, None], zN, iN)
        order = np.argsort(r1 ^ r2, axis=1, kind="stable")
        perm = np.take_along_axis(perm, order, axis=1)
    _perms_cache = perm
    return _perms_cache


_packed_cache = None


def _packed_idx() -> np.ndarray:
    global _packed_cache
    if _packed_cache is None:
        perm = _perms()
        W = np.arange(_N // 2, dtype=np.int64)
        G = W >> 4
        i = W & 15
        lo = perm[:, 32 * G + i].astype(np.int64)
        hi = perm[:, 32 * G + 16 + i].astype(np.int64)
        _packed_cache = (lo | (hi << 16)).astype(np.int32)
    return _packed_cache


def _sc_body(g_hbm, idx_hbm, o_hbm, g_buf, i_buf0, i_buf1, o_buf0, o_buf1,
             g_sem, i_sems, o_sems):
    wid = lax.axis_index("s") * _NC + lax.axis_index("c")
    slab0 = wid * _SLABS_PER_W
    g3 = g_hbm.reshape(_NSLABS, _SLAB_ROWS, _N)
    p3 = idx_hbm.reshape(_NSLABS, _SLAB_ROWS, _N // 2)
    o3 = o_hbm.reshape(_NSLABS, _SLAB_ROWS, _N)
    i_bufs = (i_buf0, i_buf1)
    o_bufs = (o_buf0, o_buf1)

    def issue_g(s):
        pltpu.async_copy(g3.at[slab0 + s], g_buf, g_sem)

    def wait_g():
        pltpu.make_async_copy(g3.at[0], g_buf, g_sem).wait()

    def issue_idx(t, slot):
        pltpu.async_copy(
            p3.at[slab0 + t // _NCHUNK, :, pl.ds((t % _NCHUNK) * _CHW, _CHW)],
            i_bufs[slot], i_sems.at[slot])

    def wait_idx(slot):
        pltpu.make_async_copy(p3.at[0, :, pl.ds(0, _CHW)], i_bufs[slot],
                              i_sems.at[slot]).wait()

    def wait_out(slot):
        pltpu.make_async_copy(o_bufs[slot], o3.at[0, :, pl.ds(0, _CH)],
                              o_sems.at[slot]).wait()

    issue_g(0)
    issue_idx(0, 0)
    issue_idx(1, 1)

    @pl.loop(0, _SLABS_PER_W)
    def _slab(s):
        wait_g()
        for k in range(_NCHUNK):
            slot = k % 2
            t = s * _NCHUNK + k
            wait_idx(slot)

            @pl.when(t >= 2)
            def _():
                wait_out(slot)

            ibuf = i_bufs[slot]
            obuf = o_bufs[slot]

            rvs = [jnp.full((_L,), r, jnp.int32) for r in range(_SLAB_ROWS)]

            @plsc.parallel_loop(0, _CHW, step=_L)
            def _gather(cw):
                for r in range(_SLAB_ROWS):
                    w = ibuf[r, pl.ds(cw, _L)]
                    a = jnp.bitwise_and(w, jnp.int32(0xFFFF))
                    b = lax.shift_right_logical(w, jnp.int32(16))
                    obuf[r, pl.ds(2 * cw, _L)] = plsc.load_gather(g_buf, [rvs[r], a])
                    obuf[r, pl.ds(2 * cw + _L, _L)] = plsc.load_gather(g_buf, [rvs[r], b])

            if k == _NCHUNK - 1:
                @pl.when(s + 1 < _SLABS_PER_W)
                def _():
                    issue_g(s + 1)

            @pl.when(t + 2 < _SLABS_PER_W * _NCHUNK)
            def _():
                issue_idx(t + 2, slot)

            pltpu.async_copy(
                obuf,
                o3.at[slab0 + s, :, pl.ds(k * _CH, _CH)],
                o_sems.at[slot])

    for slot in range(2):
        wait_out(slot)


@jax.jit
def _permute(g, idx):
    mesh = plsc.VectorSubcoreMesh(
        core_axis_name="c", subcore_axis_name="s",
        num_cores=_NC, num_subcores=_NS,
    )
    fn = pl.kernel(
        _sc_body,
        out_type=jax.ShapeDtypeStruct((_B, _N), jnp.float32),
        mesh=mesh,
        scratch_types=[
            pltpu.VMEM((_SLAB_ROWS, _N), jnp.float32),
            pltpu.VMEM((_SLAB_ROWS, _CHW), jnp.int32),
            pltpu.VMEM((_SLAB_ROWS, _CHW), jnp.int32),
            pltpu.VMEM((_SLAB_ROWS, _CH), jnp.float32),
            pltpu.VMEM((_SLAB_ROWS, _CH), jnp.float32),
            pltpu.SemaphoreType.DMA,
            pltpu.SemaphoreType.DMA((2,)),
            pltpu.SemaphoreType.DMA((2,)),
        ],
        compiler_params=pltpu.CompilerParams(
            use_tc_tiling_on_sc=True, needs_layout_passes=False),
    )
    return fn(g, idx)


def kernel(g):
    return _permute(g, jnp.asarray(_packed_idx()))


# speedup vs baseline: 1.0158x; 1.0158x over previous
import functools

import numpy as np
import jax
import jax.numpy as jnp
from jax import lax
from jax.experimental import pallas as pl
from jax.experimental.pallas import tpu as pltpu
from jax.experimental.pallas import tpu_sc as plsc

_B, _N = 4096, 8192
_NC, _NS = 2, 16
_NW = _NC * _NS
_L = 16

_SLAB_ROWS = 8
_NSLABS = _B // _SLAB_ROWS
_SLABS_PER_W = _NSLABS // _NW
_CH = 2048
_NCHUNK = _N // _CH
_CHW = _CH // 2

_perms_cache = None

_U32 = np.uint32


def _rotl(x, d):
    return (x << _U32(d)) | (x >> _U32(32 - d))


def _threefry2x32(k1, k2, x1, x2):
    rot0 = (13, 15, 26, 6)
    rot1 = (17, 29, 16, 24)
    ks0, ks1 = _U32(k1), _U32(k2)
    ks2 = ks0 ^ ks1 ^ _U32(0x1BD11BDA)
    v = [(x1 + ks0).astype(_U32), (x2 + ks1).astype(_U32)]

    def rounds(rots):
        for r in rots:
            v[0] = (v[0] + v[1]).astype(_U32)
            v[1] = v[0] ^ _rotl(v[1], r)

    rounds(rot0); v[0] = (v[0] + ks1).astype(_U32); v[1] = (v[1] + ks2 + _U32(1)).astype(_U32)
    rounds(rot1); v[0] = (v[0] + ks2).astype(_U32); v[1] = (v[1] + ks0 + _U32(2)).astype(_U32)
    rounds(rot0); v[0] = (v[0] + ks0).astype(_U32); v[1] = (v[1] + ks1 + _U32(3)).astype(_U32)
    rounds(rot1); v[0] = (v[0] + ks1).astype(_U32); v[1] = (v[1] + ks2 + _U32(4)).astype(_U32)
    rounds(rot0); v[0] = (v[0] + ks2).astype(_U32); v[1] = (v[1] + ks0 + _U32(5)).astype(_U32)
    return v[0], v[1]


def _perms() -> np.ndarray:
    global _perms_cache
    if _perms_cache is not None:
        return _perms_cache
    seed = 42
    b1, b2 = _threefry2x32(_U32(seed >> 32), _U32(seed & 0xFFFFFFFF),
                           np.zeros(_B, dtype=_U32), np.arange(_B, dtype=_U32))
    keys = np.stack([b1, b2], axis=1)

    perm = np.broadcast_to(np.arange(_N, dtype=np.int32), (_B, _N)).copy()
    num_rounds = int(np.ceil(3 * np.log(max(1, _N)) / np.log(np.iinfo(np.uint32).max)))
    z2 = np.broadcast_to(np.zeros(2, dtype=_U32), (_B, 2))
    i2 = np.broadcast_to(np.arange(2, dtype=_U32), (_B, 2))
    zN = np.broadcast_to(np.zeros(_N, dtype=_U32), (_B, _N))
    iN = np.broadcast_to(np.arange(_N, dtype=_U32), (_B, _N))
    for _ in range(num_rounds):
        s1, s2 = _threefry2x32(keys[:, 0, None], keys[:, 1, None], z2, i2)
        keys = np.stack([s1[:, 0], s2[:, 0]], axis=1)
        r1, r2 = _threefry2x32(s1[:, 1, None], s2[:, 1, None], zN, iN)
        order = np.argsort(r1 ^ r2, axis=1, kind="stable")
        perm = np.take_along_axis(perm, order, axis=1)
    _perms_cache = perm
    return _perms_cache


_packed_cache = None


def _packed_idx() -> np.ndarray:
    global _packed_cache
    if _packed_cache is None:
        perm = _perms()
        W = np.arange(_N // 2, dtype=np.int64)
        G = W >> 4
        i = W & 15
        lo = perm[:, 32 * G + i].astype(np.int64)
        hi = perm[:, 32 * G + 16 + i].astype(np.int64)
        _packed_cache = (lo | (hi << 16)).astype(np.int32)
    return _packed_cache


def _sc_body(g_hbm, idx_hbm, o_hbm, g_buf, i_buf0, i_buf1, o_buf0, o_buf1,
             g_sem, i_sems, o_sems):
    wid = lax.axis_index("s") * _NC + lax.axis_index("c")
    slab0 = wid * _SLABS_PER_W
    g3 = g_hbm.reshape(_NSLABS, _SLAB_ROWS, _N)
    p3 = idx_hbm.reshape(_NSLABS, _SLAB_ROWS, _N // 2)
    o3 = o_hbm.reshape(_NSLABS, _SLAB_ROWS, _N)
    i_bufs = (i_buf0, i_buf1)
    o_bufs = (o_buf0, o_buf1)

    def issue_g(s):
        pltpu.async_copy(g3.at[slab0 + s], g_buf, g_sem)

    def wait_g():
        pltpu.make_async_copy(g3.at[0], g_buf, g_sem).wait()

    def issue_idx(t, slot):
        pltpu.async_copy(
            p3.at[slab0 + t // _NCHUNK, :, pl.ds((t % _NCHUNK) * _CHW, _CHW)],
            i_bufs[slot], i_sems.at[slot])

    def wait_idx(slot):
        pltpu.make_async_copy(p3.at[0, :, pl.ds(0, _CHW)], i_bufs[slot],
                              i_sems.at[slot]).wait()

    def wait_out(slot):
        pltpu.make_async_copy(o_bufs[slot], o3.at[0, :, pl.ds(0, _CH)],
                              o_sems.at[slot]).wait()

    issue_g(0)
    issue_idx(0, 0)
    issue_idx(1, 1)

    @pl.loop(0, _SLABS_PER_W)
    def _slab(s):
        wait_g()
        for k in range(_NCHUNK):
            slot = k % 2
            t = s * _NCHUNK + k
            wait_idx(slot)

            @pl.when(t >= 2)
            def _():
                wait_out(slot)

            ibuf = i_bufs[slot]
            obuf = o_bufs[slot]

            rvs = [jnp.full((_L,), r, jnp.int32) for r in range(_SLAB_ROWS)]

            @plsc.parallel_loop(0, _CHW, step=_L)
            def _gather(cw):
                for r in range(_SLAB_ROWS):
                    w = ibuf[r, pl.ds(cw, _L)]
                    a = jnp.bitwise_and(w, jnp.int32(0xFFFF))
                    b = lax.shift_right_logical(w, jnp.int32(16))
                    obuf[r, pl.ds(2 * cw, _L)] = plsc.load_gather(g_buf, [rvs[r], a])
                    obuf[r, pl.ds(2 * cw + _L, _L)] = plsc.load_gather(g_buf, [rvs[r], b])

            @pl.when(t + 2 < _SLABS_PER_W * _NCHUNK)
            def _():
                issue_idx(t + 2, slot)

            pltpu.async_copy(
                obuf,
                o3.at[slab0 + s, :, pl.ds(k * _CH, _CH)],
                o_sems.at[slot])

        @pl.when(s + 1 < _SLABS_PER_W)
        def _():
            issue_g(s + 1)

    for slot in range(2):
        wait_out(slot)


@jax.jit
def _permute(g, idx):
    mesh = plsc.VectorSubcoreMesh(
        core_axis_name="c", subcore_axis_name="s",
        num_cores=_NC, num_subcores=_NS,
    )
    fn = pl.kernel(
        _sc_body,
        out_type=jax.ShapeDtypeStruct((_B, _N), jnp.float32),
        mesh=mesh,
        scratch_types=[
            pltpu.VMEM((_SLAB_ROWS, _N), jnp.float32),
            pltpu.VMEM((_SLAB_ROWS, _CHW), jnp.int32),
            pltpu.VMEM((_SLAB_ROWS, _CHW), jnp.int32),
            pltpu.VMEM((_SLAB_ROWS, _CH), jnp.float32),
            pltpu.VMEM((_SLAB_ROWS, _CH), jnp.float32),
            pltpu.SemaphoreType.DMA,
            pltpu.SemaphoreType.DMA((2,)),
            pltpu.SemaphoreType.DMA((2,)),
        ],
        compiler_params=pltpu.CompilerParams(
            use_tc_tiling_on_sc=True, needs_layout_passes=False),
    )
    return fn(g, idx)


def kernel(g):
    return _permute(g, jnp.asarray(_packed_idx()))
